# baseline (speedup 1.0000x reference)
import functools

import jax
import jax.numpy as jnp
from jax import lax
from jax.experimental import pallas as pl
from jax.experimental.pallas import tpu as pltpu
from jax.experimental.pallas import tpu_sc as plsc

T = 2048
D = 1024
E = 64
F = 512
K = 2
BM = 128
P = 12288
NB = P // BM


CH = 128
NCH = T // CH


def _router_body(x_ref, gw_ref, d0_ref, d1_ref, w1_ref, w2_ref,
                 be_ref, used_ref):
    x = x_ref[...]
    gw = gw_ref[...]
    logits = jax.lax.dot_general(x, gw, (((1,), (1,)), ((), ())),
                                 preferred_element_type=jnp.float32)
    m1 = jnp.max(logits, axis=1, keepdims=True)
    iota = jax.lax.broadcasted_iota(jnp.int32, logits.shape, 1)
    big = jnp.int32(1 << 30)
    i1 = jnp.min(jnp.where(logits == m1, iota, big), axis=1, keepdims=True)
    mask1 = iota == i1
    m2 = jnp.max(jnp.where(mask1, -jnp.inf, logits), axis=1, keepdims=True)
    i2 = jnp.min(jnp.where((logits == m2) & (~mask1), iota, big),
                 axis=1, keepdims=True)
    w1 = 1.0 / (1.0 + jnp.exp(m2 - m1))
    w1_ref[...] = w1
    w2_ref[...] = 1.0 - w1

    oh1 = (iota == i1).astype(jnp.float32)
    oh2 = (iota == i2).astype(jnp.float32)
    ohc = oh1 + oh2
    ri = jax.lax.broadcasted_iota(jnp.int32, (CH, CH), 0)
    ci = jax.lax.broadcasted_iota(jnp.int32, (CH, CH), 1)
    tril = (ci < ri).astype(jnp.float32)
    base_rows = []
    running = jnp.zeros((1, E), jnp.float32)
    for c in range(NCH):
        chunk = ohc[c * CH:(c + 1) * CH]
        excl = jnp.dot(tril, chunk, preferred_element_type=jnp.float32)
        base_rows.append(excl + running)
        running = running + jnp.sum(chunk, axis=0, keepdims=True)
    base = jnp.concatenate(base_rows, axis=0)
    counts = running

    nblk = jnp.ceil(counts / BM)
    ei = jax.lax.broadcasted_iota(jnp.int32, (E, E), 0)
    ej = jax.lax.broadcasted_iota(jnp.int32, (E, E), 1)
    triu_inc = (ei <= ej).astype(jnp.float32)
    bend = jnp.dot(nblk, triu_inc, preferred_element_type=jnp.float32)
    pstart = (bend - nblk) * BM
    d0 = jnp.sum(oh1 * (pstart + base), axis=1, keepdims=True)
    d1 = jnp.sum(oh2 * (pstart + base), axis=1, keepdims=True)
    d0_ref[...] = d0.astype(jnp.int32)
    d1_ref[...] = d1.astype(jnp.int32)

    used2d = jax.lax.slice(bend, (0, E - 1), (1, E))
    used = used2d[0, 0]
    used_ref[...] = used2d.astype(jnp.int32)
    blki = jax.lax.broadcasted_iota(jnp.int32, (NB, E), 0).astype(jnp.float32)
    be_raw = jnp.sum((bend <= blki).astype(jnp.float32), axis=1, keepdims=True)
    eidx = jax.lax.broadcasted_iota(jnp.int32, (1, E), 1).astype(jnp.float32)
    last_e = jnp.max(jnp.where(counts > 0, eidx, -1.0))
    blk1 = jax.lax.broadcasted_iota(jnp.int32, (NB, 1), 0).astype(jnp.float32)
    be = jnp.where(blk1 < used, be_raw, last_e)
    be_ref[...] = be.astype(jnp.int32)


def _run_router(x2d, gate_w):
    out_shapes = (
        jax.ShapeDtypeStruct((T, 1), jnp.int32),
        jax.ShapeDtypeStruct((T, 1), jnp.int32),
        jax.ShapeDtypeStruct((T, 1), jnp.float32),
        jax.ShapeDtypeStruct((T, 1), jnp.float32),
        jax.ShapeDtypeStruct((NB, 1), jnp.int32),
        jax.ShapeDtypeStruct((1, 1), jnp.int32),
    )
    return pl.pallas_call(_router_body, out_shape=out_shapes)(x2d, gate_w)


def _mm_body(be_ref, used_ref, x_ref, wg_ref, wu_ref, wd_ref, rw_ref, out_ref):
    b = pl.program_id(0)

    @pl.when(b < used_ref[0])
    def _():
        x = x_ref[...].astype(jnp.bfloat16)
        g = jnp.dot(x, wg_ref[0].astype(jnp.bfloat16),
                    preferred_element_type=jnp.float32)
        u = jnp.dot(x, wu_ref[0].astype(jnp.bfloat16),
                    preferred_element_type=jnp.float32)
        act = ((g * jax.nn.sigmoid(g)) * u).astype(jnp.bfloat16)
        y = jnp.dot(act, wd_ref[0].astype(jnp.bfloat16),
                    preferred_element_type=jnp.float32)
        out_ref[...] = y * rw_ref[...]


def _run_grouped_mm(xs, w_gate, w_up, w_down, rw_sorted, be, used):
    def live(b, be_ref, used_ref):
        return (jnp.minimum(b, used_ref[0] - 1), 0)

    grid_spec = pltpu.PrefetchScalarGridSpec(
        num_scalar_prefetch=2,
        grid=(NB,),
        in_specs=[
            pl.BlockSpec((BM, D), live),
            pl.BlockSpec((1, D, F), lambda b, be_ref, u: (be_ref[b], 0, 0)),
            pl.BlockSpec((1, D, F), lambda b, be_ref, u: (be_ref[b], 0, 0)),
            pl.BlockSpec((1, F, D), lambda b, be_ref, u: (be_ref[b], 0, 0)),
            pl.BlockSpec((BM, 1), live),
        ],
        out_specs=pl.BlockSpec((BM, D), live),
    )
    return pl.pallas_call(
        _mm_body,
        grid_spec=grid_spec,
        out_shape=jax.ShapeDtypeStruct((P, D), jnp.float32),
    )(be, used, xs, w_gate, w_up, w_down, rw_sorted)


NW = 32
RPW = P // NW
CHR = 64
TPW = T // NW
CHT = 32

_SC_MESH = dict(core_axis_name="c", subcore_axis_name="s")


@functools.partial(
    pl.kernel,
    mesh=plsc.VectorSubcoreMesh(**_SC_MESH),
    out_type=jax.ShapeDtypeStruct((P, D), jnp.float32),
    scratch_types=[
        pltpu.VMEM((CHR,), jnp.int32),
        pltpu.VMEM((CHR, D), jnp.float32),
        pltpu.SemaphoreType.DMA,
    ],
)
def _sc_dispatch(x_hbm, src_hbm, xs_hbm, idx_v, rows_v, sem):
    wid = lax.axis_index("s") * 2 + lax.axis_index("c")
    base = wid * RPW
    for c in range(RPW // CHR):
        off = base + c * CHR
        pltpu.sync_copy(src_hbm.at[pl.ds(off, CHR)], idx_v)
        pltpu.async_copy(x_hbm.at[idx_v], rows_v, sem).wait()
        pltpu.sync_copy(rows_v, xs_hbm.at[pl.ds(off, CHR)])


@functools.partial(
    pl.kernel,
    mesh=plsc.VectorSubcoreMesh(**_SC_MESH),
    out_type=jax.ShapeDtypeStruct((T, D), jnp.float32),
    scratch_types=[
        pltpu.VMEM((CHT,), jnp.int32),
        pltpu.VMEM((CHT,), jnp.int32),
        pltpu.VMEM((CHT, D), jnp.float32),
        pltpu.VMEM((CHT, D), jnp.float32),
        pltpu.SemaphoreType.DMA,
    ],
)
def _sc_combine(y_hbm, d0_hbm, d1_hbm, out_hbm, i0_v, i1_v, r0_v, r1_v, sem):
    wid = lax.axis_index("s") * 2 + lax.axis_index("c")
    base = wid * TPW
    nvec = D // 16
    for c in range(TPW // CHT):
        off = base + c * CHT
        pltpu.sync_copy(d0_hbm.at[pl.ds(off, CHT)], i0_v)
        pltpu.sync_copy(d1_hbm.at[pl.ds(off, CHT)], i1_v)
        pltpu.async_copy(y_hbm.at[i0_v], r0_v, sem).wait()
        pltpu.async_copy(y_hbm.at[i1_v], r1_v, sem).wait()

        def body(k, carry):
            i = k // nvec
            sl = pl.ds((k % nvec) * 16, 16)
            r0_v[i, sl] = r0_v[i, sl] + r1_v[i, sl]
            return carry

        lax.fori_loop(0, CHT * nvec, body, 0)
        pltpu.sync_copy(r0_v, out_hbm.at[pl.ds(off, CHT)])


def kernel(hidden_states, gate_w, w_gate, w_up, w_down):
    b, s, d = hidden_states.shape
    x2d = hidden_states.reshape(-1, d)

    d0, d1, w1, w2, be, used = _run_router(x2d, gate_w)

    dest = jnp.concatenate([d0, d1], axis=1).reshape(-1)
    rwflat = jnp.concatenate([w1, w2], axis=1).reshape(-1)
    src_idx = jnp.zeros(P, jnp.int32).at[dest].set(
        (jnp.arange(T * K) // K).astype(jnp.int32))
    rw_sorted = jnp.zeros((P, 1), jnp.float32).at[dest, 0].set(rwflat)

    xs = _sc_dispatch(x2d, src_idx)

    y = _run_grouped_mm(xs, w_gate, w_up, w_down, rw_sorted,
                        be.reshape(-1), used.reshape(-1))

    out = _sc_combine(y, d0.reshape(-1), d1.reshape(-1))
    return out.reshape(b, s, d)


# speedup vs baseline: 1.2583x; 1.2583x over previous
import functools

import jax
import jax.numpy as jnp
from jax import lax
from jax.experimental import pallas as pl
from jax.experimental.pallas import tpu as pltpu
from jax.experimental.pallas import tpu_sc as plsc

T = 2048
D = 1024
E = 64
F = 512
K = 2
BM = 64
P = 8192
NB = P // BM


CH = 128
NCH = T // CH


def _router_body(x_ref, gw_ref, d0_ref, d1_ref, w1_ref, w2_ref,
                 be_ref, used_ref):
    x = x_ref[...]
    gw = gw_ref[...]
    logits = jax.lax.dot_general(x, gw, (((1,), (1,)), ((), ())),
                                 preferred_element_type=jnp.float32)
    m1 = jnp.max(logits, axis=1, keepdims=True)
    iota = jax.lax.broadcasted_iota(jnp.int32, logits.shape, 1)
    big = jnp.int32(1 << 30)
    i1 = jnp.min(jnp.where(logits == m1, iota, big), axis=1, keepdims=True)
    mask1 = iota == i1
    m2 = jnp.max(jnp.where(mask1, -jnp.inf, logits), axis=1, keepdims=True)
    i2 = jnp.min(jnp.where((logits == m2) & (~mask1), iota, big),
                 axis=1, keepdims=True)
    w1 = 1.0 / (1.0 + jnp.exp(m2 - m1))
    w1_ref[...] = w1
    w2_ref[...] = 1.0 - w1

    oh1 = (iota == i1).astype(jnp.float32)
    oh2 = (iota == i2).astype(jnp.float32)
    ohc = oh1 + oh2
    ri = jax.lax.broadcasted_iota(jnp.int32, (CH, CH), 0)
    ci = jax.lax.broadcasted_iota(jnp.int32, (CH, CH), 1)
    tril = (ci < ri).astype(jnp.float32)
    base_rows = []
    running = jnp.zeros((1, E), jnp.float32)
    for c in range(NCH):
        chunk = ohc[c * CH:(c + 1) * CH]
        excl = jnp.dot(tril, chunk, preferred_element_type=jnp.float32)
        base_rows.append(excl + running)
        running = running + jnp.sum(chunk, axis=0, keepdims=True)
    base = jnp.concatenate(base_rows, axis=0)
    counts = running

    nblk = jnp.ceil(counts / BM)
    ei = jax.lax.broadcasted_iota(jnp.int32, (E, E), 0)
    ej = jax.lax.broadcasted_iota(jnp.int32, (E, E), 1)
    triu_inc = (ei <= ej).astype(jnp.float32)
    bend = jnp.dot(nblk, triu_inc, preferred_element_type=jnp.float32)
    pstart = (bend - nblk) * BM
    d0 = jnp.sum(oh1 * (pstart + base), axis=1, keepdims=True)
    d1 = jnp.sum(oh2 * (pstart + base), axis=1, keepdims=True)
    d0_ref[...] = d0.astype(jnp.int32)
    d1_ref[...] = d1.astype(jnp.int32)

    used2d = jax.lax.slice(bend, (0, E - 1), (1, E))
    used = used2d[0, 0]
    used_ref[...] = used2d.astype(jnp.int32)
    blki = jax.lax.broadcasted_iota(jnp.int32, (NB, E), 0).astype(jnp.float32)
    be_raw = jnp.sum((bend <= blki).astype(jnp.float32), axis=1, keepdims=True)
    eidx = jax.lax.broadcasted_iota(jnp.int32, (1, E), 1).astype(jnp.float32)
    last_e = jnp.max(jnp.where(counts > 0, eidx, -1.0))
    blk1 = jax.lax.broadcasted_iota(jnp.int32, (NB, 1), 0).astype(jnp.float32)
    be = jnp.where(blk1 < used, be_raw, last_e)
    be_ref[...] = be.astype(jnp.int32)


def _run_router(x2d, gate_w):
    out_shapes = (
        jax.ShapeDtypeStruct((T, 1), jnp.int32),
        jax.ShapeDtypeStruct((T, 1), jnp.int32),
        jax.ShapeDtypeStruct((T, 1), jnp.float32),
        jax.ShapeDtypeStruct((T, 1), jnp.float32),
        jax.ShapeDtypeStruct((NB, 1), jnp.int32),
        jax.ShapeDtypeStruct((1, 1), jnp.int32),
    )
    return pl.pallas_call(_router_body, out_shape=out_shapes)(x2d, gate_w)


def _mm_body(be_ref, used_ref, x_ref, wg_ref, wu_ref, wd_ref, rw_ref, out_ref):
    b = pl.program_id(0)

    @pl.when(b < used_ref[0])
    def _():
        x = x_ref[...].astype(jnp.bfloat16)
        g = jnp.dot(x, wg_ref[0].astype(jnp.bfloat16),
                    preferred_element_type=jnp.float32)
        u = jnp.dot(x, wu_ref[0].astype(jnp.bfloat16),
                    preferred_element_type=jnp.float32)
        act = ((g * jax.nn.sigmoid(g)) * u).astype(jnp.bfloat16)
        y = jnp.dot(act, wd_ref[0].astype(jnp.bfloat16),
                    preferred_element_type=jnp.float32)
        out_ref[...] = y * rw_ref[...]


def _run_grouped_mm(xs, w_gate, w_up, w_down, rw_sorted, be, used):
    def live(b, be_ref, used_ref):
        return (jnp.minimum(b, used_ref[0] - 1), 0)

    grid_spec = pltpu.PrefetchScalarGridSpec(
        num_scalar_prefetch=2,
        grid=(NB,),
        in_specs=[
            pl.BlockSpec((BM, D), live),
            pl.BlockSpec((1, D, F), lambda b, be_ref, u: (be_ref[b], 0, 0)),
            pl.BlockSpec((1, D, F), lambda b, be_ref, u: (be_ref[b], 0, 0)),
            pl.BlockSpec((1, F, D), lambda b, be_ref, u: (be_ref[b], 0, 0)),
            pl.BlockSpec((BM, 1), live),
        ],
        out_specs=pl.BlockSpec((BM, D), live),
    )
    return pl.pallas_call(
        _mm_body,
        grid_spec=grid_spec,
        out_shape=jax.ShapeDtypeStruct((P, D), jnp.float32),
    )(be, used, xs, w_gate, w_up, w_down, rw_sorted)


NW = 32
RPW = P // NW
CHR = 32
TPW = T // NW
CHT = 32

_SC_MESH = dict(core_axis_name="c", subcore_axis_name="s")


@functools.partial(
    pl.kernel,
    mesh=plsc.VectorSubcoreMesh(**_SC_MESH),
    out_type=jax.ShapeDtypeStruct((P, D), jnp.float32),
    scratch_types=[
        pltpu.VMEM((CHR,), jnp.int32),
        pltpu.VMEM((CHR,), jnp.int32),
        pltpu.VMEM((CHR, D), jnp.float32),
        pltpu.VMEM((CHR, D), jnp.float32),
        pltpu.SemaphoreType.DMA,
        pltpu.SemaphoreType.DMA,
    ],
)
def _sc_dispatch(x_hbm, src_hbm, xs_hbm, i0_v, i1_v, r0_v, r1_v, s0, s1):
    wid = lax.axis_index("s") * 2 + lax.axis_index("c")
    base = wid * RPW
    nchk = RPW // CHR
    idxs, rows, sems = (i0_v, i1_v), (r0_v, r1_v), (s0, s1)
    cps = [None, None]
    pltpu.sync_copy(src_hbm.at[pl.ds(base, CHR)], i0_v)
    cps[0] = pltpu.async_copy(x_hbm.at[i0_v], r0_v, s0)
    for c in range(nchk):
        b = c % 2
        nb = (c + 1) % 2
        if c + 1 < nchk:
            off = base + (c + 1) * CHR
            pltpu.sync_copy(src_hbm.at[pl.ds(off, CHR)], idxs[nb])
            cps[nb] = pltpu.async_copy(x_hbm.at[idxs[nb]], rows[nb], sems[nb])
        cps[b].wait()
        pltpu.sync_copy(rows[b], xs_hbm.at[pl.ds(base + c * CHR, CHR)])


@functools.partial(
    pl.kernel,
    mesh=plsc.VectorSubcoreMesh(**_SC_MESH),
    out_type=jax.ShapeDtypeStruct((T, D), jnp.float32),
    scratch_types=[
        pltpu.VMEM((CHT,), jnp.int32),
        pltpu.VMEM((CHT,), jnp.int32),
        pltpu.VMEM((CHT, D), jnp.float32),
        pltpu.VMEM((CHT, D), jnp.float32),
        pltpu.SemaphoreType.DMA,
        pltpu.SemaphoreType.DMA,
    ],
)
def _sc_combine(y_hbm, d0_hbm, d1_hbm, out_hbm, i0_v, i1_v, r0_v, r1_v,
                sem, sem2):
    wid = lax.axis_index("s") * 2 + lax.axis_index("c")
    base = wid * TPW
    nvec = D // 16
    for c in range(TPW // CHT):
        off = base + c * CHT
        pltpu.sync_copy(d0_hbm.at[pl.ds(off, CHT)], i0_v)
        pltpu.sync_copy(d1_hbm.at[pl.ds(off, CHT)], i1_v)
        cp0 = pltpu.async_copy(y_hbm.at[i0_v], r0_v, sem)
        cp1 = pltpu.async_copy(y_hbm.at[i1_v], r1_v, sem2)
        cp0.wait()
        cp1.wait()

        def body(k, carry):
            i = k // nvec
            sl = pl.ds((k % nvec) * 16, 16)
            r0_v[i, sl] = r0_v[i, sl] + r1_v[i, sl]
            return carry

        lax.fori_loop(0, CHT * nvec, body, 0)
        pltpu.sync_copy(r0_v, out_hbm.at[pl.ds(off, CHT)])


def kernel(hidden_states, gate_w, w_gate, w_up, w_down):
    b, s, d = hidden_states.shape
    x2d = hidden_states.reshape(-1, d)

    d0, d1, w1, w2, be, used = _run_router(x2d, gate_w)

    dest = jnp.concatenate([d0, d1], axis=1).reshape(-1)
    rwflat = jnp.concatenate([w1, w2], axis=1).reshape(-1)
    src_idx = jnp.zeros(P, jnp.int32).at[dest].set(
        (jnp.arange(T * K) // K).astype(jnp.int32))
    rw_sorted = jnp.zeros((P, 1), jnp.float32).at[dest, 0].set(rwflat)

    xs = _sc_dispatch(x2d, src_idx)

    y = _run_grouped_mm(xs, w_gate, w_up, w_down, rw_sorted,
                        be.reshape(-1), used.reshape(-1))

    out = _sc_combine(y, d0.reshape(-1), d1.reshape(-1))
    return out.reshape(b, s, d)


# speedup vs baseline: 1.2602x; 1.0015x over previous
import functools

import jax
import jax.numpy as jnp
from jax import lax
from jax.experimental import pallas as pl
from jax.experimental.pallas import tpu as pltpu
from jax.experimental.pallas import tpu_sc as plsc

T = 2048
D = 1024
E = 64
F = 512
K = 2
BM = 64
P = 8192
NB = P // BM


CH = 128
NCH = T // CH


def _router_body(x_ref, gw_ref, d0_ref, d1_ref, w1_ref, w2_ref,
                 be_ref, used_ref):
    x = x_ref[...]
    gw = gw_ref[...]
    logits = jax.lax.dot_general(x, gw, (((1,), (1,)), ((), ())),
                                 preferred_element_type=jnp.float32)
    m1 = jnp.max(logits, axis=1, keepdims=True)
    iota = jax.lax.broadcasted_iota(jnp.int32, logits.shape, 1)
    big = jnp.int32(1 << 30)
    i1 = jnp.min(jnp.where(logits == m1, iota, big), axis=1, keepdims=True)
    mask1 = iota == i1
    m2 = jnp.max(jnp.where(mask1, -jnp.inf, logits), axis=1, keepdims=True)
    i2 = jnp.min(jnp.where((logits == m2) & (~mask1), iota, big),
                 axis=1, keepdims=True)
    w1 = 1.0 / (1.0 + jnp.exp(m2 - m1))
    w1_ref[...] = w1
    w2_ref[...] = 1.0 - w1

    oh1 = (iota == i1).astype(jnp.float32)
    oh2 = (iota == i2).astype(jnp.float32)
    ohc = oh1 + oh2
    ri = jax.lax.broadcasted_iota(jnp.int32, (CH, CH), 0)
    ci = jax.lax.broadcasted_iota(jnp.int32, (CH, CH), 1)
    tril = (ci < ri).astype(jnp.float32)
    base_rows = []
    running = jnp.zeros((1, E), jnp.float32)
    for c in range(NCH):
        chunk = ohc[c * CH:(c + 1) * CH]
        excl = jnp.dot(tril, chunk, preferred_element_type=jnp.float32)
        base_rows.append(excl + running)
        running = running + jnp.sum(chunk, axis=0, keepdims=True)
    base = jnp.concatenate(base_rows, axis=0)
    counts = running

    nblk = jnp.ceil(counts / BM)
    ei = jax.lax.broadcasted_iota(jnp.int32, (E, E), 0)
    ej = jax.lax.broadcasted_iota(jnp.int32, (E, E), 1)
    triu_inc = (ei <= ej).astype(jnp.float32)
    bend = jnp.dot(nblk, triu_inc, preferred_element_type=jnp.float32)
    pstart = (bend - nblk) * BM
    d0 = jnp.sum(oh1 * (pstart + base), axis=1, keepdims=True)
    d1 = jnp.sum(oh2 * (pstart + base), axis=1, keepdims=True)
    d0_ref[...] = d0.astype(jnp.int32)
    d1_ref[...] = d1.astype(jnp.int32)

    used2d = jax.lax.slice(bend, (0, E - 1), (1, E))
    used = used2d[0, 0]
    used_ref[...] = used2d.astype(jnp.int32)
    blki = jax.lax.broadcasted_iota(jnp.int32, (NB, E), 0).astype(jnp.float32)
    be_raw = jnp.sum((bend <= blki).astype(jnp.float32), axis=1, keepdims=True)
    eidx = jax.lax.broadcasted_iota(jnp.int32, (1, E), 1).astype(jnp.float32)
    last_e = jnp.max(jnp.where(counts > 0, eidx, -1.0))
    blk1 = jax.lax.broadcasted_iota(jnp.int32, (NB, 1), 0).astype(jnp.float32)
    be = jnp.where(blk1 < used, be_raw, last_e)
    be_ref[...] = be.astype(jnp.int32)


def _run_router(x2d, gate_w):
    out_shapes = (
        jax.ShapeDtypeStruct((T, 1), jnp.int32),
        jax.ShapeDtypeStruct((T, 1), jnp.int32),
        jax.ShapeDtypeStruct((T, 1), jnp.float32),
        jax.ShapeDtypeStruct((T, 1), jnp.float32),
        jax.ShapeDtypeStruct((NB, 1), jnp.int32),
        jax.ShapeDtypeStruct((1, 1), jnp.int32),
    )
    return pl.pallas_call(_router_body, out_shape=out_shapes)(x2d, gate_w)


def _mm_body(be_ref, used_ref, x_ref, wg_ref, wu_ref, wd_ref, rw_ref, out_ref):
    b = pl.program_id(0)

    @pl.when(b < used_ref[0])
    def _():
        x = x_ref[...].astype(jnp.bfloat16)
        g = jnp.dot(x, wg_ref[0].astype(jnp.bfloat16),
                    preferred_element_type=jnp.float32)
        u = jnp.dot(x, wu_ref[0].astype(jnp.bfloat16),
                    preferred_element_type=jnp.float32)
        act = ((g * jax.nn.sigmoid(g)) * u).astype(jnp.bfloat16)
        y = jnp.dot(act, wd_ref[0].astype(jnp.bfloat16),
                    preferred_element_type=jnp.float32)
        out_ref[...] = y * rw_ref[...]


def _run_grouped_mm(xs, w_gate, w_up, w_down, rw_sorted, be, used):
    def live(b, be_ref, used_ref):
        return (jnp.minimum(b, used_ref[0] - 1), 0)

    grid_spec = pltpu.PrefetchScalarGridSpec(
        num_scalar_prefetch=2,
        grid=(NB,),
        in_specs=[
            pl.BlockSpec((BM, D), live),
            pl.BlockSpec((1, D, F), lambda b, be_ref, u: (be_ref[b], 0, 0)),
            pl.BlockSpec((1, D, F), lambda b, be_ref, u: (be_ref[b], 0, 0)),
            pl.BlockSpec((1, F, D), lambda b, be_ref, u: (be_ref[b], 0, 0)),
            pl.BlockSpec((BM, 1), live),
        ],
        out_specs=pl.BlockSpec((BM, D), live),
    )
    return pl.pallas_call(
        _mm_body,
        grid_spec=grid_spec,
        out_shape=jax.ShapeDtypeStruct((P, D), jnp.float32),
    )(be, used, xs, w_gate, w_up, w_down, rw_sorted)


NW = 32
RPW = P // NW
CHR = 32
TPW = T // NW
CHT = 32

_SC_MESH = dict(core_axis_name="c", subcore_axis_name="s")


@functools.partial(
    pl.kernel,
    mesh=plsc.VectorSubcoreMesh(**_SC_MESH),
    out_type=jax.ShapeDtypeStruct((P, D), jnp.float32),
    scratch_types=[
        pltpu.VMEM((RPW,), jnp.int32),
        pltpu.VMEM((CHR, D), jnp.float32),
        pltpu.VMEM((CHR, D), jnp.float32),
        pltpu.SemaphoreType.DMA,
        pltpu.SemaphoreType.DMA,
        pltpu.SemaphoreType.DMA,
        pltpu.SemaphoreType.DMA,
    ],
)
def _sc_dispatch(x_hbm, src_hbm, xs_hbm, idx_v, r0_v, r1_v, g0, g1, w0, w1):
    wid = lax.axis_index("s") * 2 + lax.axis_index("c")
    base = wid * RPW
    nchk = RPW // CHR
    rows, gsem, wsem = (r0_v, r1_v), (g0, g1), (w0, w1)
    gcp = [None, None]
    wcp = [None, None]
    pltpu.sync_copy(src_hbm.at[pl.ds(base, RPW)], idx_v)
    gcp[0] = pltpu.async_copy(x_hbm.at[idx_v.at[pl.ds(0, CHR)]], r0_v, g0)
    for c in range(nchk):
        b = c % 2
        nb = (c + 1) % 2
        if c + 1 < nchk:
            if wcp[nb] is not None:
                wcp[nb].wait()
            gcp[nb] = pltpu.async_copy(
                x_hbm.at[idx_v.at[pl.ds((c + 1) * CHR, CHR)]],
                rows[nb], gsem[nb])
        gcp[b].wait()
        wcp[b] = pltpu.async_copy(
            rows[b], xs_hbm.at[pl.ds(base + c * CHR, CHR)], wsem[b])
    wcp[(nchk - 1) % 2].wait()
    wcp[nchk % 2].wait()


@functools.partial(
    pl.kernel,
    mesh=plsc.VectorSubcoreMesh(**_SC_MESH),
    out_type=jax.ShapeDtypeStruct((T, D), jnp.float32),
    scratch_types=[
        pltpu.VMEM((CHT,), jnp.int32),
        pltpu.VMEM((CHT,), jnp.int32),
        pltpu.VMEM((CHT, D), jnp.float32),
        pltpu.VMEM((CHT, D), jnp.float32),
        pltpu.SemaphoreType.DMA,
        pltpu.SemaphoreType.DMA,
    ],
)
def _sc_combine(y_hbm, d0_hbm, d1_hbm, out_hbm, i0_v, i1_v, r0_v, r1_v,
                sem, sem2):
    wid = lax.axis_index("s") * 2 + lax.axis_index("c")
    base = wid * TPW
    nvec = D // 16
    for c in range(TPW // CHT):
        off = base + c * CHT
        pltpu.sync_copy(d0_hbm.at[pl.ds(off, CHT)], i0_v)
        pltpu.sync_copy(d1_hbm.at[pl.ds(off, CHT)], i1_v)
        cp0 = pltpu.async_copy(y_hbm.at[i0_v], r0_v, sem)
        cp1 = pltpu.async_copy(y_hbm.at[i1_v], r1_v, sem2)
        cp0.wait()
        cp1.wait()

        def body(k, carry):
            i = k // nvec
            sl = pl.ds((k % nvec) * 16, 16)
            r0_v[i, sl] = r0_v[i, sl] + r1_v[i, sl]
            return carry

        lax.fori_loop(0, CHT * nvec, body, 0)
        pltpu.sync_copy(r0_v, out_hbm.at[pl.ds(off, CHT)])


def kernel(hidden_states, gate_w, w_gate, w_up, w_down):
    b, s, d = hidden_states.shape
    x2d = hidden_states.reshape(-1, d)

    d0, d1, w1, w2, be, used = _run_router(x2d, gate_w)

    dest = jnp.concatenate([d0, d1], axis=1).reshape(-1)
    rwflat = jnp.concatenate([w1, w2], axis=1).reshape(-1)
    src_idx = jnp.zeros(P, jnp.int32).at[dest].set(
        (jnp.arange(T * K) // K).astype(jnp.int32))
    rw_sorted = jnp.zeros((P, 1), jnp.float32).at[dest, 0].set(rwflat)

    xs = _sc_dispatch(x2d, src_idx)

    y = _run_grouped_mm(xs, w_gate, w_up, w_down, rw_sorted,
                        be.reshape(-1), used.reshape(-1))

    out = _sc_combine(y, d0.reshape(-1), d1.reshape(-1))
    return out.reshape(b, s, d)
